# baseline (speedup 1.0000x reference)
import jax
import jax.numpy as jnp
from jax import lax
from jax.experimental import pallas as pl
from jax.experimental.pallas import tpu as pltpu
from jax.experimental.pallas import tpu_sc as plsc

B, NC = 2, 8
H, W = 512, 512
C, H1, W1 = 128, 256, 256
N = H1 * W1
TEMP = 10.0
LOSS_WEIGHT = 0.1
EPS = 1e-8

NUM_CORES, NUM_SUBCORES, LANES = 2, 16, 16
NUM_TILES = NUM_CORES * NUM_SUBCORES
ROWS_PER_TILE = H1 // NUM_SUBCORES
PIX_PER_TILE = ROWS_PER_TILE * W1
NKEY = NC * NC

PCHUNK = 64
NPCHUNK = PIX_PER_TILE // PCHUNK
NBUF = 4
GROWS = 4
NGRP = ROWS_PER_TILE // GROWS
TRBLK = 2048


def _tr_body(x_ref, o_ref):
    o_ref[...] = x_ref[0].T


def _tr_call(con_flat):
    return pl.pallas_call(
        _tr_body,
        grid=(B, N // TRBLK),
        in_specs=[pl.BlockSpec((1, C, TRBLK), lambda b, p: (b, 0, p))],
        out_specs=pl.BlockSpec((TRBLK, C), lambda b, p: (b * (N // TRBLK) + p, 0)),
        out_shape=jax.ShapeDtypeStruct((B * N, C), jnp.float32),
    )(con_flat)


def _sc_body(cls_hbm, lab_hbm, cont_hbm, accs_hbm, cnts_hbm,
             clsbuf, labbuf, keybuf, acc, cntv, conbuf, spacc,
             gsem0, gsem1, gsem2, gsem3, ssem0, ssem1, ssem2, ssem3,
             csem, lsem):
    gsems = (gsem0, gsem1, gsem2, gsem3)
    ssems = (ssem0, ssem1, ssem2, ssem3)
    cid = lax.axis_index("c")
    sid = lax.axis_index("s")
    b = cid
    wid = cid * NUM_SUBCORES + sid

    zero = jnp.zeros((LANES,), jnp.float32)
    iota = lax.iota(jnp.int32, LANES)
    col_even = iota * 2

    @pl.loop(0, NKEY)
    def _zacc(r):
        for l8 in range(C // LANES):
            acc[r, pl.ds(l8 * LANES, LANES)] = zero
    for l4 in range(NKEY // LANES):
        cntv[pl.ds(l4 * LANES, LANES)] = zero

    @pl.when(sid == 0)
    def _():
        pltpu.sync_copy(acc, spacc)

    row0 = sid * ROWS_PER_TILE

    def grp_copy(g, par):
        orow = row0 + g * GROWS
        ccopy = pltpu.make_async_copy(
            cls_hbm.at[b, :, pl.ds(orow, GROWS), 0, :], clsbuf.at[par], csem)
        lcopy = pltpu.make_async_copy(
            lab_hbm.at[b, 0, pl.ds(orow, GROWS), 0, :], labbuf.at[par], lsem)
        return ccopy, lcopy

    for cp in grp_copy(0, 0):
        cp.start()

    @pl.loop(0, NGRP)
    def _grp(g):
        par = lax.rem(g, 2)
        for cp in grp_copy(g, par):
            cp.wait()

        @pl.when(g + 1 < NGRP)
        def _():
            for cp in grp_copy(g + 1, 1 - lax.rem(g, 2)):
                cp.start()

        for par_s in range(2):

            @pl.when(par == par_s)
            def _():
                for ro in range(GROWS):
                    for gcol in range(W1 // LANES):
                        cidx = col_even + (32 * gcol)
                        ro_s = jnp.full_like(cidx, ro)
                        labv = plsc.load_gather(labbuf.at[par_s], [ro_s, cidx])
                        best = plsc.load_gather(
                            clsbuf.at[par_s], [jnp.zeros_like(cidx), ro_s, cidx])
                        bi = jnp.zeros((LANES,), jnp.int32)
                        for ch in range(1, NC):
                            v = plsc.load_gather(
                                clsbuf.at[par_s],
                                [jnp.full_like(cidx, ch), ro_s, cidx])
                            m = v > best
                            best = jnp.where(m, v, best)
                            bi = jnp.where(m, jnp.int32(ch), bi)
                        key = bi * NC + labv
                        pos_row = g * (GROWS * W1 // PCHUNK) + (
                            (ro * W1 + gcol * LANES) // PCHUNK)
                        pos_col = (ro * W1 + gcol * LANES) % PCHUNK
                        keybuf[pos_row, pl.ds(pos_col, LANES)] = key
                        plsc.addupdate_scatter(
                            cntv, [key], jnp.ones((LANES,), jnp.float32))

    base_row = b * N + sid * PIX_PER_TILE

    def chunk_gather(j, q):
        return pltpu.make_async_copy(
            cont_hbm.at[pl.ds(base_row + j * PCHUNK, PCHUNK)],
            conbuf.at[q], gsems[q])

    def chunk_scatter(j, q):
        return pltpu.make_async_copy(
            conbuf.at[q], spacc.at[keybuf.at[j]], ssems[q])

    for q0 in range(NBUF):
        chunk_gather(q0, q0).start()
    plsc.subcore_barrier()

    @pl.loop(0, NPCHUNK, step=NBUF)
    def _outer(jj):
        for q in range(NBUF):
            j = jj + q
            chunk_gather(j, q).wait()
            pltpu.async_copy(conbuf.at[q], spacc.at[keybuf.at[j]], ssems[q],
                             add=True)
            nj = j + NBUF

            @pl.when(nj < NPCHUNK)
            def _():
                chunk_scatter(j, q).wait()
                chunk_gather(nj, q).start()

    for q in range(NBUF):
        chunk_scatter(NPCHUNK - NBUF + q, q).wait()

    plsc.subcore_barrier()

    @pl.when(sid == 0)
    def _():
        pltpu.sync_copy(spacc, accs_hbm.at[b])

    pltpu.sync_copy(cntv, cnts_hbm.at[wid])


def _sc_call(cls_score, label_i, con_t):
    fn = pl.kernel(
        _sc_body,
        out_type=[
            jax.ShapeDtypeStruct((B, NKEY, C), jnp.float32),
            jax.ShapeDtypeStruct((NUM_TILES, NKEY), jnp.float32),
        ],
        mesh=plsc.VectorSubcoreMesh(core_axis_name="c", subcore_axis_name="s"),
        compiler_params=pltpu.CompilerParams(needs_layout_passes=False),
        scratch_types=[
            pltpu.VMEM((2, NC, GROWS, W), jnp.float32),
            pltpu.VMEM((2, GROWS, W), jnp.int32),
            pltpu.VMEM((NPCHUNK, PCHUNK), jnp.int32),
            pltpu.VMEM((NKEY, C), jnp.float32),
            pltpu.VMEM((NKEY,), jnp.float32),
            pltpu.VMEM((NBUF, PCHUNK, C), jnp.float32),
            pltpu.VMEM_SHARED((NKEY, C), jnp.float32),
        ] + [pltpu.SemaphoreType.DMA] * 10,
    )
    return fn(cls_score, label_i, con_t)


def _cls_sum_body(x_ref, o_ref):
    @pl.when(pl.program_id(0) == 0)
    def _():
        o_ref[0, 0] = jnp.float32(0.0)

    o_ref[0, 0] += jnp.sum(x_ref[...])


def _cls_sum_call(cls_score):
    return pl.pallas_call(
        _cls_sum_body,
        grid=(B * NC,),
        in_specs=[pl.BlockSpec((1, 1, H, W), lambda i: (i // NC, i % NC, 0, 0))],
        out_specs=pl.BlockSpec(memory_space=pltpu.SMEM),
        out_shape=jax.ShapeDtypeStruct((1, 1), jnp.float32),
    )(cls_score)


def _final_body(accs_ref, cnts_ref, clssum_ref, o_ref):
    A = accs_ref[...]
    ct32 = cnts_ref[...]
    ctf = jnp.sum(ct32.reshape(B, NUM_SUBCORES, NKEY), axis=1)
    con_sum = jnp.sum(A)

    A4 = A.reshape(B, NC, NC, C)
    ct = ctf.reshape(B, NC, NC)
    jj = lax.broadcasted_iota(jnp.int32, (NC, NC), 0)
    kk = lax.broadcasted_iota(jnp.int32, (NC, NC), 1)
    eye = (jj == kk)
    eyef = eye.astype(jnp.float32)

    cnt_tt = jnp.sum(ct * eyef[None], axis=2)
    ttsum = jnp.sum(A4 * eyef[None, :, :, None], axis=2)
    tt_mean = ttsum / jnp.maximum(cnt_tt, 1.0)[:, :, None]
    cr = A4 / jnp.maximum(ct, 1.0)[..., None]
    pos = jnp.broadcast_to(tt_mean[:, None, :, :], cr.shape)
    neg = jnp.where(
        jnp.broadcast_to((cnt_tt > 0)[:, :, None, None], cr.shape),
        jnp.broadcast_to(tt_mean[:, :, None, :], cr.shape),
        cr,
    )

    def nrm(x):
        return x / (jnp.sqrt(jnp.sum(x * x, axis=-1, keepdims=True)) + EPS)

    cn, pn, ngn = nrm(cr), nrm(pos), nrm(neg)
    sp = jnp.sum(cn * pn, axis=-1) * TEMP
    sn = jnp.sum(cn * ngn, axis=-1) * TEMP
    mx = jnp.maximum(sp, sn)
    lse = mx + jnp.log(jnp.exp(sp - mx) + jnp.exp(sn - mx))
    per_region = lse - sp

    presentf = (jnp.sum(ct, axis=1) > 0).astype(jnp.float32)
    validf = ((ct > 0).astype(jnp.float32)
              * (cnt_tt > 0).astype(jnp.float32)[:, None, :]
              * presentf[:, :, None]
              * (1.0 - eyef)[None])
    nvalid = jnp.sum(validf)
    loss = LOSS_WEIGHT * jnp.sum(per_region * validf) / jnp.maximum(nvalid, 1.0)
    fallback = (-clssum_ref[0, 0] + con_sum) * 1e-16
    o_ref[0, 0] = jnp.where(nvalid > 0, loss, fallback)


def _final_call(accs, cnts, cls_sum):
    return pl.pallas_call(
        _final_body,
        in_specs=[
            pl.BlockSpec(memory_space=pltpu.VMEM),
            pl.BlockSpec(memory_space=pltpu.VMEM),
            pl.BlockSpec(memory_space=pltpu.SMEM),
        ],
        out_specs=pl.BlockSpec(memory_space=pltpu.SMEM),
        out_shape=jax.ShapeDtypeStruct((1, 1), jnp.float32),
    )(accs, cnts, cls_sum)


def kernel(cls_score, label, con_seg_logit):
    label_i = label.astype(jnp.int32).reshape(B, 1, H1, 2, W)
    cls_v = cls_score.reshape(B, NC, H1, 2, W)
    con_flat = con_seg_logit.reshape(B, C, N)
    con_t = _tr_call(con_flat)
    accs, cnts = _sc_call(cls_v, label_i, con_t)
    cls_sum = _cls_sum_call(cls_score)
    out = _final_call(accs, cnts, cls_sum)
    return out[0, 0]


# speedup vs baseline: 1.0492x; 1.0492x over previous
import jax
import jax.numpy as jnp
from jax import lax
from jax.experimental import pallas as pl
from jax.experimental.pallas import tpu as pltpu
from jax.experimental.pallas import tpu_sc as plsc

B, NC = 2, 8
H, W = 512, 512
C, H1, W1 = 128, 256, 256
N = H1 * W1
TEMP = 10.0
LOSS_WEIGHT = 0.1
EPS = 1e-8

NUM_CORES, NUM_SUBCORES, LANES = 2, 16, 16
NUM_TILES = NUM_CORES * NUM_SUBCORES
ROWS_PER_TILE = H1 // NUM_SUBCORES
PIX_PER_TILE = ROWS_PER_TILE * W1
NKEY = NC * NC

PCHUNK = 64
NPCHUNK = PIX_PER_TILE // PCHUNK
NBUF = 4
GROWS = 4
NGRP = ROWS_PER_TILE // GROWS
TRBLK = 2048


def _tr_body(x_ref, o_ref):
    o_ref[...] = x_ref[0].T


def _tr_call(con_flat):
    return pl.pallas_call(
        _tr_body,
        grid=(B, N // TRBLK),
        in_specs=[pl.BlockSpec((1, C, TRBLK), lambda b, p: (b, 0, p))],
        out_specs=pl.BlockSpec((TRBLK, C), lambda b, p: (b * (N // TRBLK) + p, 0)),
        out_shape=jax.ShapeDtypeStruct((B * N, C), jnp.float32),
    )(con_flat)


def _sca_body(cls_hbm, lab_hbm, keys_hbm, cnts_hbm,
              clsbuf, labbuf, keybuf, cntv, csem, lsem):
    cid = lax.axis_index("c")
    sid = lax.axis_index("s")
    b = cid
    wid = cid * NUM_SUBCORES + sid

    zero = jnp.zeros((LANES,), jnp.float32)
    iota = lax.iota(jnp.int32, LANES)
    col_even = iota * 2

    for l4 in range(NKEY // LANES):
        cntv[pl.ds(l4 * LANES, LANES)] = zero

    row0 = sid * ROWS_PER_TILE

    def grp_copy(g, par):
        orow = row0 + g * GROWS
        ccopy = pltpu.make_async_copy(
            cls_hbm.at[b, :, pl.ds(orow, GROWS), 0, :], clsbuf.at[par], csem)
        lcopy = pltpu.make_async_copy(
            lab_hbm.at[b, 0, pl.ds(orow, GROWS), 0, :], labbuf.at[par], lsem)
        return ccopy, lcopy

    for cp in grp_copy(0, 0):
        cp.start()

    @pl.loop(0, NGRP)
    def _grp(g):
        par = lax.rem(g, 2)
        for cp in grp_copy(g, par):
            cp.wait()

        @pl.when(g + 1 < NGRP)
        def _():
            for cp in grp_copy(g + 1, 1 - lax.rem(g, 2)):
                cp.start()

        for par_s in range(2):

            @pl.when(par == par_s)
            def _():
                for ro in range(GROWS):
                    for gcol in range(W1 // LANES):
                        cidx = col_even + (32 * gcol)
                        ro_s = jnp.full_like(cidx, ro)
                        labv = plsc.load_gather(labbuf.at[par_s], [ro_s, cidx])
                        best = plsc.load_gather(
                            clsbuf.at[par_s], [jnp.zeros_like(cidx), ro_s, cidx])
                        bi = jnp.zeros((LANES,), jnp.int32)
                        for ch in range(1, NC):
                            v = plsc.load_gather(
                                clsbuf.at[par_s],
                                [jnp.full_like(cidx, ch), ro_s, cidx])
                            m = v > best
                            best = jnp.where(m, v, best)
                            bi = jnp.where(m, jnp.int32(ch), bi)
                        key = bi * NC + labv
                        pos_row = g * (GROWS * W1 // PCHUNK) + (
                            (ro * W1 + gcol * LANES) // PCHUNK)
                        pos_col = (ro * W1 + gcol * LANES) % PCHUNK
                        keybuf[pos_row, pl.ds(pos_col, LANES)] = key
                        plsc.addupdate_scatter(
                            cntv, [key], jnp.ones((LANES,), jnp.float32))

    pltpu.sync_copy(keybuf, keys_hbm.at[wid])
    pltpu.sync_copy(cntv, cnts_hbm.at[wid])


def _sca_call(cls_v, label_v):
    fn = pl.kernel(
        _sca_body,
        out_type=[
            jax.ShapeDtypeStruct((NUM_TILES, NPCHUNK, PCHUNK), jnp.int32),
            jax.ShapeDtypeStruct((NUM_TILES, NKEY), jnp.float32),
        ],
        mesh=plsc.VectorSubcoreMesh(core_axis_name="c", subcore_axis_name="s"),
        compiler_params=pltpu.CompilerParams(needs_layout_passes=False),
        scratch_types=[
            pltpu.VMEM((2, NC, GROWS, W), jnp.float32),
            pltpu.VMEM((2, GROWS, W), jnp.int32),
            pltpu.VMEM((NPCHUNK, PCHUNK), jnp.int32),
            pltpu.VMEM((NKEY,), jnp.float32),
            pltpu.SemaphoreType.DMA,
            pltpu.SemaphoreType.DMA,
        ],
    )
    return fn(cls_v, label_v)


def _scb_body(cont_hbm, keys_hbm, accs_hbm,
              keybuf, acc, conbuf, spacc,
              gsem0, gsem1, gsem2, gsem3, ssem0, ssem1, ssem2, ssem3, ksem):
    gsems = (gsem0, gsem1, gsem2, gsem3)
    ssems = (ssem0, ssem1, ssem2, ssem3)
    cid = lax.axis_index("c")
    sid = lax.axis_index("s")
    b = cid
    wid = cid * NUM_SUBCORES + sid

    kload = pltpu.make_async_copy(keys_hbm.at[wid], keybuf, ksem)
    kload.start()

    zero = jnp.zeros((LANES,), jnp.float32)

    @pl.loop(0, NKEY)
    def _zacc(r):
        for l8 in range(C // LANES):
            acc[r, pl.ds(l8 * LANES, LANES)] = zero

    @pl.when(sid == 0)
    def _():
        pltpu.sync_copy(acc, spacc)

    base_row = b * N + sid * PIX_PER_TILE

    def chunk_gather(j, q):
        return pltpu.make_async_copy(
            cont_hbm.at[pl.ds(base_row + j * PCHUNK, PCHUNK)],
            conbuf.at[q], gsems[q])

    def chunk_scatter(j, q):
        return pltpu.make_async_copy(
            conbuf.at[q], spacc.at[keybuf.at[j]], ssems[q])

    for q0 in range(NBUF):
        chunk_gather(q0, q0).start()
    kload.wait()
    plsc.subcore_barrier()

    @pl.loop(0, NPCHUNK, step=NBUF)
    def _outer(jj):
        for q in range(NBUF):
            j = jj + q
            chunk_gather(j, q).wait()
            pltpu.async_copy(conbuf.at[q], spacc.at[keybuf.at[j]], ssems[q],
                             add=True)
            nj = j + NBUF

            @pl.when(nj < NPCHUNK)
            def _():
                chunk_scatter(j, q).wait()
                chunk_gather(nj, q).start()

    for q in range(NBUF):
        chunk_scatter(NPCHUNK - NBUF + q, q).wait()

    plsc.subcore_barrier()

    @pl.when(sid == 0)
    def _():
        pltpu.sync_copy(spacc, accs_hbm.at[b])


def _scb_call(con_t, keys):
    fn = pl.kernel(
        _scb_body,
        out_type=[
            jax.ShapeDtypeStruct((B, NKEY, C), jnp.float32),
        ],
        mesh=plsc.VectorSubcoreMesh(core_axis_name="c", subcore_axis_name="s"),
        compiler_params=pltpu.CompilerParams(needs_layout_passes=False),
        scratch_types=[
            pltpu.VMEM((NPCHUNK, PCHUNK), jnp.int32),
            pltpu.VMEM((NKEY, C), jnp.float32),
            pltpu.VMEM((NBUF, PCHUNK, C), jnp.float32),
            pltpu.VMEM_SHARED((NKEY, C), jnp.float32),
        ] + [pltpu.SemaphoreType.DMA] * 9,
    )
    return fn(con_t, keys)


def _cls_sum_body(x_ref, o_ref):
    @pl.when(pl.program_id(0) == 0)
    def _():
        o_ref[0, 0] = jnp.float32(0.0)

    o_ref[0, 0] += jnp.sum(x_ref[...])


def _cls_sum_call(cls_score):
    return pl.pallas_call(
        _cls_sum_body,
        grid=(B * NC,),
        in_specs=[pl.BlockSpec((1, 1, H, W), lambda i: (i // NC, i % NC, 0, 0))],
        out_specs=pl.BlockSpec(memory_space=pltpu.SMEM),
        out_shape=jax.ShapeDtypeStruct((1, 1), jnp.float32),
    )(cls_score)


def _final_body(accs_ref, cnts_ref, clssum_ref, o_ref):
    A = accs_ref[...]
    ct32 = cnts_ref[...]
    ctf = jnp.sum(ct32.reshape(B, NUM_SUBCORES, NKEY), axis=1)
    con_sum = jnp.sum(A)

    A4 = A.reshape(B, NC, NC, C)
    ct = ctf.reshape(B, NC, NC)
    jj = lax.broadcasted_iota(jnp.int32, (NC, NC), 0)
    kk = lax.broadcasted_iota(jnp.int32, (NC, NC), 1)
    eye = (jj == kk)
    eyef = eye.astype(jnp.float32)

    cnt_tt = jnp.sum(ct * eyef[None], axis=2)
    ttsum = jnp.sum(A4 * eyef[None, :, :, None], axis=2)
    tt_mean = ttsum / jnp.maximum(cnt_tt, 1.0)[:, :, None]
    cr = A4 / jnp.maximum(ct, 1.0)[..., None]
    pos = jnp.broadcast_to(tt_mean[:, None, :, :], cr.shape)
    neg = jnp.where(
        jnp.broadcast_to((cnt_tt > 0)[:, :, None, None], cr.shape),
        jnp.broadcast_to(tt_mean[:, :, None, :], cr.shape),
        cr,
    )

    def nrm(x):
        return x / (jnp.sqrt(jnp.sum(x * x, axis=-1, keepdims=True)) + EPS)

    cn, pn, ngn = nrm(cr), nrm(pos), nrm(neg)
    sp = jnp.sum(cn * pn, axis=-1) * TEMP
    sn = jnp.sum(cn * ngn, axis=-1) * TEMP
    mx = jnp.maximum(sp, sn)
    lse = mx + jnp.log(jnp.exp(sp - mx) + jnp.exp(sn - mx))
    per_region = lse - sp

    presentf = (jnp.sum(ct, axis=1) > 0).astype(jnp.float32)
    validf = ((ct > 0).astype(jnp.float32)
              * (cnt_tt > 0).astype(jnp.float32)[:, None, :]
              * presentf[:, :, None]
              * (1.0 - eyef)[None])
    nvalid = jnp.sum(validf)
    loss = LOSS_WEIGHT * jnp.sum(per_region * validf) / jnp.maximum(nvalid, 1.0)
    fallback = (-clssum_ref[0, 0] + con_sum) * 1e-16
    o_ref[0, 0] = jnp.where(nvalid > 0, loss, fallback)


def _final_call(accs, cnts, cls_sum):
    return pl.pallas_call(
        _final_body,
        in_specs=[
            pl.BlockSpec(memory_space=pltpu.VMEM),
            pl.BlockSpec(memory_space=pltpu.VMEM),
            pl.BlockSpec(memory_space=pltpu.SMEM),
        ],
        out_specs=pl.BlockSpec(memory_space=pltpu.SMEM),
        out_shape=jax.ShapeDtypeStruct((1, 1), jnp.float32),
    )(accs, cnts, cls_sum)


def kernel(cls_score, label, con_seg_logit):
    label_i = label.astype(jnp.int32).reshape(B, 1, H1, 2, W)
    cls_v = cls_score.reshape(B, NC, H1, 2, W)
    con_flat = con_seg_logit.reshape(B, C, N)
    keys, cnts = _sca_call(cls_v, label_i)
    con_t = _tr_call(con_flat)
    (accs,) = _scb_call(con_t, keys)
    cls_sum = _cls_sum_call(cls_score)
    out = _final_call(accs, cnts, cls_sum)
    return out[0, 0]
